# baseline (speedup 1.0000x reference)
import functools

import jax
import jax.numpy as jnp
from jax import lax
from jax.experimental import pallas as pl
from jax.experimental.pallas import tpu as pltpu
from jax.experimental.pallas import tpu_sc as plsc

_CHUNK = 8
_NBUF = 5
_LANES = 16


def kernel(mask, embed_weight):
    B, L = mask.shape
    D = embed_weight.shape[-1]
    info = plsc.get_sparse_core_info()
    n_cores, n_sub = info.num_cores, info.num_subcores
    n_workers = n_cores * n_sub
    rows_per_w = L // n_workers
    n_chunks = B // _CHUNK

    mesh = plsc.VectorSubcoreMesh(core_axis_name="c", subcore_axis_name="s")

    @functools.partial(
        pl.kernel,
        out_type=jax.ShapeDtypeStruct((L, B, D), embed_weight.dtype),
        mesh=mesh,
        scratch_types=[
            pltpu.VMEM((L // n_workers, D), embed_weight.dtype),
            pltpu.VMEM((_NBUF, _CHUNK, D), embed_weight.dtype),
            pltpu.SemaphoreType.DMA((_NBUF,)),
        ],
    )
    def sc_embed(table_hbm, out_hbm, rows_v, rep_v, sem):
        wid = lax.axis_index("s") * n_cores + lax.axis_index("c")
        for j in range(rows_per_w):
            pltpu.sync_copy(
                table_hbm.at[pl.ds(wid + j * n_workers, 1)],
                rows_v.at[pl.ds(j, 1)],
            )

        def fire(i, args):
            l, p = args
            pltpu.make_async_copy(
                rep_v.at[p], out_hbm.at[l, pl.ds(i * _CHUNK, _CHUNK)], sem.at[p]
            ).start()
            return args

        def drain(i, args):
            l, p = args
            pltpu.make_async_copy(
                rep_v.at[p], out_hbm.at[l, pl.ds(i * _CHUNK, _CHUNK)], sem.at[p]
            ).wait()
            return args

        for j in range(rows_per_w):
            p = j % _NBUF
            if j >= _NBUF:
                lax.fori_loop(0, n_chunks, drain, (wid + (j - _NBUF) * n_workers, p))
            vecs = [
                rows_v[j, pl.ds(v * _LANES, _LANES)] for v in range(D // _LANES)
            ]

            def fill(i, p2, vecs=vecs):
                for v in range(D // _LANES):
                    rep_v[p2, i, pl.ds(v * _LANES, _LANES)] = vecs[v]
                return p2

            lax.fori_loop(0, _CHUNK, fill, p)
            lax.fori_loop(0, n_chunks, fire, (wid + j * n_workers, p))

        for j in range(max(rows_per_w - _NBUF, 0), rows_per_w):
            lax.fori_loop(0, n_chunks, drain, (wid + j * n_workers, j % _NBUF))

    return sc_embed(embed_weight)


# speedup vs baseline: 1.0134x; 1.0134x over previous
import functools

import jax
import jax.numpy as jnp
from jax import lax
from jax.experimental import pallas as pl
from jax.experimental.pallas import tpu as pltpu
from jax.experimental.pallas import tpu_sc as plsc

_CHUNK = 16
_NBUF = 5
_LANES = 16


def kernel(mask, embed_weight):
    B, L = mask.shape
    D = embed_weight.shape[-1]
    info = plsc.get_sparse_core_info()
    n_cores, n_sub = info.num_cores, info.num_subcores
    n_workers = n_cores * n_sub
    rows_per_w = L // n_workers
    n_chunks = B // _CHUNK

    mesh = plsc.VectorSubcoreMesh(core_axis_name="c", subcore_axis_name="s")

    @functools.partial(
        pl.kernel,
        out_type=jax.ShapeDtypeStruct((L, B, D), embed_weight.dtype),
        mesh=mesh,
        scratch_types=[
            pltpu.VMEM((L // n_workers, D), embed_weight.dtype),
            pltpu.VMEM((_NBUF, _CHUNK, D), embed_weight.dtype),
            pltpu.SemaphoreType.DMA((_NBUF,)),
        ],
    )
    def sc_embed(table_hbm, out_hbm, rows_v, rep_v, sem):
        wid = lax.axis_index("s") * n_cores + lax.axis_index("c")
        for j in range(rows_per_w):
            pltpu.sync_copy(
                table_hbm.at[pl.ds(wid + j * n_workers, 1)],
                rows_v.at[pl.ds(j, 1)],
            )

        def fire(i, args):
            l, p = args
            pltpu.make_async_copy(
                rep_v.at[p], out_hbm.at[l, pl.ds(i * _CHUNK, _CHUNK)], sem.at[p]
            ).start()
            return args

        def drain(i, args):
            l, p = args
            pltpu.make_async_copy(
                rep_v.at[p], out_hbm.at[l, pl.ds(i * _CHUNK, _CHUNK)], sem.at[p]
            ).wait()
            return args

        for j in range(rows_per_w):
            p = j % _NBUF
            if j >= _NBUF:
                lax.fori_loop(0, n_chunks, drain, (wid + (j - _NBUF) * n_workers, p))
            vecs = [
                rows_v[j, pl.ds(v * _LANES, _LANES)] for v in range(D // _LANES)
            ]

            def fill(i, p2, vecs=vecs):
                for v in range(D // _LANES):
                    rep_v[p2, i, pl.ds(v * _LANES, _LANES)] = vecs[v]
                return p2

            lax.fori_loop(0, _CHUNK, fill, p)
            lax.fori_loop(0, n_chunks, fire, (wid + j * n_workers, p))

        for j in range(max(rows_per_w - _NBUF, 0), rows_per_w):
            lax.fori_loop(0, n_chunks, drain, (wid + j * n_workers, j % _NBUF))

    return sc_embed(embed_weight)
